# baseline (speedup 1.0000x reference)
import functools

import jax
import jax.numpy as jnp
from jax import lax
from jax.experimental import pallas as pl
from jax.experimental.pallas import tpu as pltpu
from jax.experimental.pallas import tpu_sc as plsc

B = 4096
L = 50
UD = 64
CW = 256
NI = 1024
BN_SCALE = (1.0 + 1e-3) ** -0.5
NW = 32
RPW = B // NW
HPASS = RPW // 2
USER_PAIRS = 50000

_SC_PARAMS = pltpu.CompilerParams(needs_layout_passes=False,
                                  use_tc_tiling_on_sc=False)


def _combine_body(it_ref, ct_ref, cl_ref, b16_ref,
                  out_ref, hi_ref, lo_ref):
    cl = cl_ref[...]
    iota = lax.broadcasted_iota(jnp.int32, (NI, 64), 1)
    oh = (cl == iota).astype(jnp.float32)
    cate_part = jnp.dot(oh, ct_ref[...], preferred_element_type=jnp.float32,
                        precision=lax.Precision.HIGHEST)
    out_ref[...] = jnp.concatenate(
        [it_ref[...], cate_part, b16_ref[...],
         jnp.zeros((NI, CW - 144), jnp.float32)], axis=1)
    c128 = jnp.concatenate([it_ref[...], cate_part], axis=1)
    hi = c128.astype(jnp.bfloat16)
    hi_ref[...] = hi
    lo_ref[...] = (c128 - hi.astype(jnp.float32)).astype(jnp.bfloat16)


_combine = pl.pallas_call(
    _combine_body,
    out_shape=[
        jax.ShapeDtypeStruct((NI, CW), jnp.float32),
        jax.ShapeDtypeStruct((NI, 128), jnp.bfloat16),
        jax.ShapeDtypeStruct((NI, 128), jnp.bfloat16),
    ],
)


def _sc_hist_body(histf_hbm, item_hbm, comb_hbm, countsf_out, ijoin_out,
                  histf_v, cntf_v, iidx_v, irows_v, sem, sem2):
    wid = lax.axis_index("s") * 2 + lax.axis_index("c")
    base = wid * RPW
    pltpu.sync_copy(item_hbm.at[pl.ds(base, RPW)], iidx_v)
    cp_i = pltpu.async_copy(comb_hbm.at[iidx_v], irows_v, sem2)
    pltpu.sync_copy(histf_hbm.at[pl.ds(base * L, RPW * L)], histf_v)
    zeros16 = jnp.zeros((16,), jnp.float32)
    ones16 = jnp.ones((16,), jnp.float32)
    iota16 = lax.broadcasted_iota(jnp.int32, (16,), 0)
    i50 = iota16 * L
    icnt = iota16 * NI
    for p in range(2):
        def zero_body(i, _):
            for u in range(64):
                cntf_v[pl.ds(i * 1024 + u * 16, 16)] = zeros16
            return 0
        lax.fori_loop(0, HPASS * NI // 1024, zero_body, 0)

        def scat_body(j, _):
            for g in range(HPASS // 16):
                row_off = p * HPASS + g * 16
                hv = plsc.load_gather(histf_v, [i50 + (row_off * L + j)])
                plsc.addupdate_scatter(cntf_v, [icnt + (hv + g * 16 * NI)],
                                       ones16)
            return 0
        lax.fori_loop(0, L, scat_body, 0)
        pltpu.async_copy(
            cntf_v,
            countsf_out.at[pl.ds((base + p * HPASS) * NI, HPASS * NI)],
            sem).wait()
    cp_i.wait()
    pltpu.sync_copy(irows_v, ijoin_out.at[pl.ds(base, RPW)])


def _sc_hist(histf, item, combined):
    mesh = plsc.VectorSubcoreMesh(core_axis_name="c", subcore_axis_name="s")
    f = functools.partial(
        pl.kernel,
        out_type=(
            jax.ShapeDtypeStruct((B * NI,), jnp.float32),
            jax.ShapeDtypeStruct((B, CW), jnp.float32),
        ),
        mesh=mesh,
        compiler_params=_SC_PARAMS,
        scratch_types=[
            pltpu.VMEM((RPW * L,), jnp.int32),
            pltpu.VMEM((HPASS * NI,), jnp.float32),
            pltpu.VMEM((RPW,), jnp.int32),
            pltpu.VMEM((RPW, CW), jnp.float32),
            pltpu.SemaphoreType.DMA,
            pltpu.SemaphoreType.DMA,
        ],
    )(_sc_hist_body)
    return f(histf, item, combined)


def _sc_gather_body(user_hbm, utable_hbm, uemb_out, uidx_v, uidx2_v,
                    urows_v, sem):
    wid = lax.axis_index("s") * 2 + lax.axis_index("c")
    base = wid * RPW
    pltpu.sync_copy(user_hbm.at[pl.ds(base, RPW)], uidx_v)
    for i in range(RPW // 16):
        uidx2_v[pl.ds(i * 16, 16)] = lax.shift_right_logical(
            uidx_v[pl.ds(i * 16, 16)], 1)
    pltpu.async_copy(utable_hbm.at[uidx2_v], urows_v, sem).wait()
    pltpu.sync_copy(urows_v, uemb_out.at[pl.ds(base, RPW)])


def _sc_gather(user, user_table2):
    mesh = plsc.VectorSubcoreMesh(core_axis_name="c", subcore_axis_name="s")
    f = functools.partial(
        pl.kernel,
        out_type=jax.ShapeDtypeStruct((B, 128), jnp.float32),
        mesh=mesh,
        compiler_params=_SC_PARAMS,
        scratch_types=[
            pltpu.VMEM((RPW,), jnp.int32),
            pltpu.VMEM((RPW,), jnp.int32),
            pltpu.VMEM((RPW, 128), jnp.float32),
            pltpu.SemaphoreType.DMA,
        ],
    )(_sc_gather_body)
    return f(user, user_table2)


def _mlp_body(uemb2_ref, ucol_ref, ijoin_ref, cntf_ref, len_ref, hi_ref,
              lo_ref, hW_ref, hb_ref, w1_ref, b1_ref, w2_ref, b2_ref,
              w3_ref, b3_ref, out_ref, logit_ref):
    cnt_bf = cntf_ref[...].reshape(512, NI).astype(jnp.bfloat16)
    hist_sum = (jnp.dot(cnt_bf, hi_ref[...], preferred_element_type=jnp.float32)
                + jnp.dot(cnt_bf, lo_ref[...],
                          preferred_element_type=jnp.float32))
    lc = len_ref[...]
    scale = (lc > 0.0).astype(jnp.float32) / lc
    hist_avg = hist_sum * scale
    hist_hid = jnp.dot(hist_avg * BN_SCALE, hW_ref[...],
                       preferred_element_type=jnp.float32) + hb_ref[...].reshape(1, 128)
    ijoin = ijoin_ref[...]
    uemb2 = uemb2_ref[...]
    odd = (ucol_ref[...] & 1) == 1
    uemb = jnp.where(odd, uemb2[:, UD:], uemb2[:, :UD])
    s = jnp.dot(uemb * BN_SCALE, w1_ref[:UD, :],
                preferred_element_type=jnp.float32)
    s += jnp.dot(ijoin[:, :128] * BN_SCALE, w1_ref[UD:UD + 128, :],
                 preferred_element_type=jnp.float32)
    s += jnp.dot(hist_hid * BN_SCALE, w1_ref[UD + 128:, :],
                 preferred_element_type=jnp.float32)
    h1 = jax.nn.sigmoid(s + b1_ref[...].reshape(1, 80))
    h2 = jax.nn.sigmoid(jnp.dot(h1, w2_ref[...],
                                preferred_element_type=jnp.float32)
                        + b2_ref[...].reshape(1, 40))
    out = (jnp.dot(h2, w3_ref[...], preferred_element_type=jnp.float32)
           + b3_ref[...].reshape(1, 1) + ijoin[:, 128:129])
    out_ref[...] = out
    logit_ref[...] = jax.nn.sigmoid(out)


def _mlp(uemb2, ucol, ijoin, countsf, lencol, comb_hi, comb_lo, hist_W,
         hist_b, fc1_W, fc1_b, fc2_W, fc2_b, fc3_W, fc3_b):
    BB = 512
    grid = B // BB
    blk = lambda r, c: pl.BlockSpec((BB, c), lambda i: (i, 0))
    full = lambda r, c: pl.BlockSpec((r, c), lambda i: (0, 0))
    vec = lambda n: pl.BlockSpec((n,), lambda i: (0,))
    return pl.pallas_call(
        _mlp_body,
        grid=(grid,),
        in_specs=[
            blk(B, 128), blk(B, 1), blk(B, CW),
            pl.BlockSpec((BB * NI,), lambda i: (i,)), blk(B, 1),
            full(NI, 128), full(NI, 128), full(128, 128), vec(128),
            full(320, 80), vec(80), full(80, 40), vec(40), full(40, 1),
            vec(1),
        ],
        out_specs=[blk(B, 1), blk(B, 1)],
        out_shape=[
            jax.ShapeDtypeStruct((B, 1), jnp.float32),
            jax.ShapeDtypeStruct((B, 1), jnp.float32),
        ],
    )(uemb2, ucol, ijoin, countsf, lencol, comb_hi, comb_lo, hist_W, hist_b,
      fc1_W, fc1_b, fc2_W, fc2_b, fc3_W, fc3_b)


def kernel(user, item, history, length, user_table, item_table, cate_table,
           item_bias, cate_list, hist_W, hist_b, fc1_W, fc1_b, fc2_W, fc2_b,
           fc3_W, fc3_b):
    itp = jnp.pad(item_table, ((0, NI - 1000), (0, 0)))
    clp = jnp.pad(cate_list, (0, NI - 1000)).reshape(NI, 1)
    b16 = jnp.pad(item_bias.reshape(-1, 1), ((0, NI - 1000), (0, 15)))
    combined, comb_hi, comb_lo = _combine(itp, cate_table, clp, b16)

    countsf, ijoin = _sc_hist(history.reshape(-1), item, combined)
    uemb2 = _sc_gather(user, user_table.reshape(USER_PAIRS, 128))

    lencol = length.reshape(B, 1).astype(jnp.float32)
    ucol = user.reshape(B, 1)
    out2, logit2 = _mlp(uemb2, ucol, ijoin, countsf, lencol, comb_hi,
                        comb_lo, hist_W, hist_b, fc1_W, fc1_b, fc2_W, fc2_b,
                        fc3_W, fc3_b)
    return out2[:, 0], logit2[:, 0]


# speedup vs baseline: 1.0027x; 1.0027x over previous
import functools

import jax
import jax.numpy as jnp
from jax import lax
from jax.experimental import pallas as pl
from jax.experimental.pallas import tpu as pltpu
from jax.experimental.pallas import tpu_sc as plsc

B = 4096
L = 50
UD = 64
CW = 256
NI = 1024
BN_SCALE = (1.0 + 1e-3) ** -0.5
NW = 32
RPW = B // NW
HPASS = RPW // 2
USER_PAIRS = 50000

_SC_PARAMS = pltpu.CompilerParams(needs_layout_passes=False,
                                  use_tc_tiling_on_sc=False)


def _combine_body(it_ref, ct_ref, cl_ref, b16_ref,
                  out_ref, hi_ref, lo_ref):
    cl = cl_ref[...]
    iota = lax.broadcasted_iota(jnp.int32, (NI, 64), 1)
    oh = (cl == iota).astype(jnp.float32)
    cate_part = jnp.dot(oh, ct_ref[...], preferred_element_type=jnp.float32,
                        precision=lax.Precision.HIGHEST)
    comb = jnp.concatenate(
        [it_ref[...], cate_part, b16_ref[...],
         jnp.zeros((NI, CW - 144), jnp.float32)], axis=1)
    out_ref[...] = comb.reshape(NI * CW)
    c128 = jnp.concatenate([it_ref[...], cate_part], axis=1)
    hi = c128.astype(jnp.bfloat16)
    hi_ref[...] = hi
    lo_ref[...] = (c128 - hi.astype(jnp.float32)).astype(jnp.bfloat16)


_combine = pl.pallas_call(
    _combine_body,
    out_shape=[
        jax.ShapeDtypeStruct((NI * CW,), jnp.float32),
        jax.ShapeDtypeStruct((NI, 128), jnp.bfloat16),
        jax.ShapeDtypeStruct((NI, 128), jnp.bfloat16),
    ],
)


def _sc_hist_body(histf_hbm, item_hbm, comb_hbm, countsf_out, ijoin_out,
                  histf_v, cntf_v, iidx_v, irows_v, sem, sem2):
    wid = lax.axis_index("s") * 2 + lax.axis_index("c")
    base = wid * RPW
    pltpu.sync_copy(item_hbm.at[pl.ds(base, RPW)], iidx_v)
    cp_i = pltpu.async_copy(comb_hbm.at[iidx_v], irows_v, sem2)
    pltpu.sync_copy(histf_hbm.at[pl.ds(base * L, RPW * L)], histf_v)
    zeros16 = jnp.zeros((16,), jnp.float32)
    ones16 = jnp.ones((16,), jnp.float32)
    iota16 = lax.broadcasted_iota(jnp.int32, (16,), 0)
    i50 = iota16 * L
    icnt = iota16 * NI
    for p in range(2):
        def zero_body(i, _):
            for u in range(64):
                cntf_v[pl.ds(i * 1024 + u * 16, 16)] = zeros16
            return 0
        lax.fori_loop(0, HPASS * NI // 1024, zero_body, 0)

        def scat_body(j, _):
            for g in range(HPASS // 16):
                row_off = p * HPASS + g * 16
                hv = plsc.load_gather(histf_v, [i50 + (row_off * L + j)])
                plsc.addupdate_scatter(cntf_v, [icnt + (hv + g * 16 * NI)],
                                       ones16)
            return 0
        lax.fori_loop(0, L, scat_body, 0)
        pltpu.async_copy(
            cntf_v,
            countsf_out.at[pl.ds((base + p * HPASS) * NI, HPASS * NI)],
            sem).wait()
    cp_i.wait()
    pltpu.sync_copy(irows_v, ijoin_out.at[pl.ds(base, RPW)])


def _sc_hist(histf, item, combined):
    mesh = plsc.VectorSubcoreMesh(core_axis_name="c", subcore_axis_name="s")
    f = functools.partial(
        pl.kernel,
        out_type=(
            jax.ShapeDtypeStruct((B * NI,), jnp.float32),
            jax.ShapeDtypeStruct((B, CW), jnp.float32),
        ),
        mesh=mesh,
        compiler_params=_SC_PARAMS,
        scratch_types=[
            pltpu.VMEM((RPW * L,), jnp.int32),
            pltpu.VMEM((HPASS * NI,), jnp.float32),
            pltpu.VMEM((RPW,), jnp.int32),
            pltpu.VMEM((RPW, CW), jnp.float32),
            pltpu.SemaphoreType.DMA,
            pltpu.SemaphoreType.DMA,
        ],
    )(_sc_hist_body)
    return f(histf, item, combined)


def _sc_gather_body(user_hbm, utable_hbm, uemb_out, uidx_v, uidx2_v,
                    urows_v, sem):
    wid = lax.axis_index("s") * 2 + lax.axis_index("c")
    base = wid * RPW
    pltpu.sync_copy(user_hbm.at[pl.ds(base, RPW)], uidx_v)
    for i in range(RPW // 16):
        uidx2_v[pl.ds(i * 16, 16)] = lax.shift_right_logical(
            uidx_v[pl.ds(i * 16, 16)], 1)
    pltpu.async_copy(utable_hbm.at[uidx2_v], urows_v, sem).wait()
    pltpu.sync_copy(urows_v, uemb_out.at[pl.ds(base, RPW)])


def _sc_gather(user, user_table2):
    mesh = plsc.VectorSubcoreMesh(core_axis_name="c", subcore_axis_name="s")
    f = functools.partial(
        pl.kernel,
        out_type=jax.ShapeDtypeStruct((B, 128), jnp.float32),
        mesh=mesh,
        compiler_params=_SC_PARAMS,
        scratch_types=[
            pltpu.VMEM((RPW,), jnp.int32),
            pltpu.VMEM((RPW,), jnp.int32),
            pltpu.VMEM((RPW, 128), jnp.float32),
            pltpu.SemaphoreType.DMA,
        ],
    )(_sc_gather_body)
    return f(user, user_table2)


def _mlp_body(uemb2_ref, ucol_ref, ijoin_ref, cntf_ref, len_ref, hi_ref,
              lo_ref, hW_ref, hb_ref, w1_ref, b1_ref, w2_ref, b2_ref,
              w3_ref, b3_ref, out_ref, logit_ref):
    cnt_bf = cntf_ref[...].reshape(512, NI).astype(jnp.bfloat16)
    hist_sum = (jnp.dot(cnt_bf, hi_ref[...], preferred_element_type=jnp.float32)
                + jnp.dot(cnt_bf, lo_ref[...],
                          preferred_element_type=jnp.float32))
    lc = len_ref[...]
    scale = (lc > 0.0).astype(jnp.float32) / lc
    hist_avg = hist_sum * scale
    hist_hid = jnp.dot(hist_avg * BN_SCALE, hW_ref[...],
                       preferred_element_type=jnp.float32) + hb_ref[...].reshape(1, 128)
    ijoin = ijoin_ref[...]
    uemb2 = uemb2_ref[...]
    odd = (ucol_ref[...] & 1) == 1
    uemb = jnp.where(odd, uemb2[:, UD:], uemb2[:, :UD])
    s = jnp.dot(uemb * BN_SCALE, w1_ref[:UD, :],
                preferred_element_type=jnp.float32)
    s += jnp.dot(ijoin[:, :128] * BN_SCALE, w1_ref[UD:UD + 128, :],
                 preferred_element_type=jnp.float32)
    s += jnp.dot(hist_hid * BN_SCALE, w1_ref[UD + 128:, :],
                 preferred_element_type=jnp.float32)
    h1 = jax.nn.sigmoid(s + b1_ref[...].reshape(1, 80))
    h2 = jax.nn.sigmoid(jnp.dot(h1, w2_ref[...],
                                preferred_element_type=jnp.float32)
                        + b2_ref[...].reshape(1, 40))
    out = (jnp.dot(h2, w3_ref[...], preferred_element_type=jnp.float32)
           + b3_ref[...].reshape(1, 1) + ijoin[:, 128:129])
    out_ref[...] = out
    logit_ref[...] = jax.nn.sigmoid(out)


def _mlp(uemb2, ucol, ijoin, countsf, lencol, comb_hi, comb_lo, hist_W,
         hist_b, fc1_W, fc1_b, fc2_W, fc2_b, fc3_W, fc3_b):
    BB = 512
    grid = B // BB
    blk = lambda r, c: pl.BlockSpec((BB, c), lambda i: (i, 0))
    full = lambda r, c: pl.BlockSpec((r, c), lambda i: (0, 0))
    vec = lambda n: pl.BlockSpec((n,), lambda i: (0,))
    return pl.pallas_call(
        _mlp_body,
        grid=(grid,),
        in_specs=[
            blk(B, 128), blk(B, 1), blk(B, CW),
            pl.BlockSpec((BB * NI,), lambda i: (i,)), blk(B, 1),
            full(NI, 128), full(NI, 128), full(128, 128), vec(128),
            full(320, 80), vec(80), full(80, 40), vec(40), full(40, 1),
            vec(1),
        ],
        out_specs=[blk(B, 1), blk(B, 1)],
        out_shape=[
            jax.ShapeDtypeStruct((B, 1), jnp.float32),
            jax.ShapeDtypeStruct((B, 1), jnp.float32),
        ],
    )(uemb2, ucol, ijoin, countsf, lencol, comb_hi, comb_lo, hist_W, hist_b,
      fc1_W, fc1_b, fc2_W, fc2_b, fc3_W, fc3_b)


def kernel(user, item, history, length, user_table, item_table, cate_table,
           item_bias, cate_list, hist_W, hist_b, fc1_W, fc1_b, fc2_W, fc2_b,
           fc3_W, fc3_b):
    itp = jnp.pad(item_table, ((0, NI - 1000), (0, 0)))
    clp = jnp.pad(cate_list, (0, NI - 1000)).reshape(NI, 1)
    b16 = jnp.pad(item_bias.reshape(-1, 1), ((0, NI - 1000), (0, 15)))
    combf, comb_hi, comb_lo = _combine(itp, cate_table, clp, b16)
    combined = combf.reshape(NI, CW)

    countsf, ijoin = _sc_hist(history.reshape(-1), item, combined)
    uemb2 = _sc_gather(user, user_table.reshape(USER_PAIRS, 128))

    lencol = length.reshape(B, 1).astype(jnp.float32)
    ucol = user.reshape(B, 1)
    out2, logit2 = _mlp(uemb2, ucol, ijoin, countsf, lencol, comb_hi,
                        comb_lo, hist_W, hist_b, fc1_W, fc1_b, fc2_W, fc2_b,
                        fc3_W, fc3_b)
    return out2[:, 0], logit2[:, 0]


# speedup vs baseline: 1.0165x; 1.0138x over previous
import functools

import jax
import jax.numpy as jnp
from jax import lax
from jax.experimental import pallas as pl
from jax.experimental.pallas import tpu as pltpu
from jax.experimental.pallas import tpu_sc as plsc

B = 4096
L = 50
UD = 64
CW = 256
NI = 1024
BN_SCALE = (1.0 + 1e-3) ** -0.5
NW = 32
RPW = B // NW
HPASS = RPW // 2
USER_PAIRS = 50000
MLP_BB = 1024

_SC_PARAMS = pltpu.CompilerParams(needs_layout_passes=False,
                                  use_tc_tiling_on_sc=False)


def _combine_body(it_ref, ct_ref, cl_ref, b16_ref,
                  out_ref, hi_ref, lo_ref):
    cl = cl_ref[...]
    iota = lax.broadcasted_iota(jnp.int32, (NI, 64), 1)
    oh = (cl == iota).astype(jnp.float32)
    cate_part = jnp.dot(oh, ct_ref[...], preferred_element_type=jnp.float32,
                        precision=lax.Precision.HIGHEST)
    comb = jnp.concatenate(
        [it_ref[...], cate_part, b16_ref[...],
         jnp.zeros((NI, CW - 144), jnp.float32)], axis=1)
    out_ref[...] = comb.reshape(NI * CW)
    c128 = jnp.concatenate([it_ref[...], cate_part], axis=1)
    hi = c128.astype(jnp.bfloat16)
    hi_ref[...] = hi
    lo_ref[...] = (c128 - hi.astype(jnp.float32)).astype(jnp.bfloat16)


_combine = pl.pallas_call(
    _combine_body,
    out_shape=[
        jax.ShapeDtypeStruct((NI * CW,), jnp.float32),
        jax.ShapeDtypeStruct((NI, 128), jnp.bfloat16),
        jax.ShapeDtypeStruct((NI, 128), jnp.bfloat16),
    ],
)


def _sc_hist_body(histf_hbm, item_hbm, comb_hbm, countsf_out, ijoin_out,
                  histf_v, cntf_v, iidx_v, irows_v, sem, sem2):
    wid = lax.axis_index("s") * 2 + lax.axis_index("c")
    base = wid * RPW
    pltpu.sync_copy(item_hbm.at[pl.ds(base, RPW)], iidx_v)
    cp_i = pltpu.async_copy(comb_hbm.at[iidx_v], irows_v, sem2)
    pltpu.sync_copy(histf_hbm.at[pl.ds(base * L, RPW * L)], histf_v)
    zeros16 = jnp.zeros((16,), jnp.float32)
    ones16 = jnp.ones((16,), jnp.float32)
    iota16 = lax.broadcasted_iota(jnp.int32, (16,), 0)
    i50 = iota16 * L
    icnt = iota16 * NI
    for p in range(2):
        def zero_body(i, _):
            for u in range(64):
                cntf_v[pl.ds(i * 1024 + u * 16, 16)] = zeros16
            return 0
        lax.fori_loop(0, HPASS * NI // 1024, zero_body, 0)

        def scat_body(j, _):
            for g in range(HPASS // 16):
                row_off = p * HPASS + g * 16
                hv = plsc.load_gather(histf_v, [i50 + (row_off * L + j)])
                plsc.addupdate_scatter(cntf_v, [icnt + (hv + g * 16 * NI)],
                                       ones16)
            return 0
        lax.fori_loop(0, L, scat_body, 0)
        pltpu.async_copy(
            cntf_v,
            countsf_out.at[pl.ds((base + p * HPASS) * NI, HPASS * NI)],
            sem).wait()
    cp_i.wait()
    pltpu.sync_copy(irows_v, ijoin_out.at[pl.ds(base, RPW)])


def _sc_hist(histf, item, combined):
    mesh = plsc.VectorSubcoreMesh(core_axis_name="c", subcore_axis_name="s")
    f = functools.partial(
        pl.kernel,
        out_type=(
            jax.ShapeDtypeStruct((B * NI,), jnp.float32),
            jax.ShapeDtypeStruct((B, CW), jnp.float32),
        ),
        mesh=mesh,
        compiler_params=_SC_PARAMS,
        scratch_types=[
            pltpu.VMEM((RPW * L,), jnp.int32),
            pltpu.VMEM((HPASS * NI,), jnp.float32),
            pltpu.VMEM((RPW,), jnp.int32),
            pltpu.VMEM((RPW, CW), jnp.float32),
            pltpu.SemaphoreType.DMA,
            pltpu.SemaphoreType.DMA,
        ],
    )(_sc_hist_body)
    return f(histf, item, combined)


def _sc_gather_body(user_hbm, utable_hbm, uemb_out, uidx_v, uidx2_v,
                    urows_v, sem):
    wid = lax.axis_index("s") * 2 + lax.axis_index("c")
    base = wid * RPW
    pltpu.sync_copy(user_hbm.at[pl.ds(base, RPW)], uidx_v)
    for i in range(RPW // 16):
        uidx2_v[pl.ds(i * 16, 16)] = lax.shift_right_logical(
            uidx_v[pl.ds(i * 16, 16)], 1)
    pltpu.async_copy(utable_hbm.at[uidx2_v], urows_v, sem).wait()
    pltpu.sync_copy(urows_v, uemb_out.at[pl.ds(base, RPW)])


def _sc_gather(user, user_table2):
    mesh = plsc.VectorSubcoreMesh(core_axis_name="c", subcore_axis_name="s")
    f = functools.partial(
        pl.kernel,
        out_type=jax.ShapeDtypeStruct((B, 128), jnp.float32),
        mesh=mesh,
        compiler_params=_SC_PARAMS,
        scratch_types=[
            pltpu.VMEM((RPW,), jnp.int32),
            pltpu.VMEM((RPW,), jnp.int32),
            pltpu.VMEM((RPW, 128), jnp.float32),
            pltpu.SemaphoreType.DMA,
        ],
    )(_sc_gather_body)
    return f(user, user_table2)


def _mlp_body(uemb2_ref, ucol_ref, ijoin_ref, cntf_ref, len_ref, hi_ref,
              lo_ref, hW_ref, hb_ref, w1_ref, b1_ref, w2_ref, b2_ref,
              w3_ref, b3_ref, out_ref, logit_ref):
    cnt_bf = cntf_ref[...].reshape(MLP_BB, NI).astype(jnp.bfloat16)
    hist_sum = (jnp.dot(cnt_bf, hi_ref[...], preferred_element_type=jnp.float32)
                + jnp.dot(cnt_bf, lo_ref[...],
                          preferred_element_type=jnp.float32))
    lc = len_ref[...]
    scale = (lc > 0.0).astype(jnp.float32) / lc
    hist_avg = hist_sum * scale
    hist_hid = jnp.dot(hist_avg * BN_SCALE, hW_ref[...],
                       preferred_element_type=jnp.float32) + hb_ref[...].reshape(1, 128)
    ijoin = ijoin_ref[...]
    uemb2 = uemb2_ref[...]
    odd = (ucol_ref[...] & 1) == 1
    uemb = jnp.where(odd, uemb2[:, UD:], uemb2[:, :UD])
    s = jnp.dot(uemb * BN_SCALE, w1_ref[:UD, :],
                preferred_element_type=jnp.float32)
    s += jnp.dot(ijoin[:, :128] * BN_SCALE, w1_ref[UD:UD + 128, :],
                 preferred_element_type=jnp.float32)
    s += jnp.dot(hist_hid * BN_SCALE, w1_ref[UD + 128:, :],
                 preferred_element_type=jnp.float32)
    h1 = jax.nn.sigmoid(s + b1_ref[...].reshape(1, 80))
    h2 = jax.nn.sigmoid(jnp.dot(h1, w2_ref[...],
                                preferred_element_type=jnp.float32)
                        + b2_ref[...].reshape(1, 40))
    out = (jnp.dot(h2, w3_ref[...], preferred_element_type=jnp.float32)
           + b3_ref[...].reshape(1, 1) + ijoin[:, 128:129])
    out_ref[...] = out
    logit_ref[...] = jax.nn.sigmoid(out)


def _mlp(uemb2, ucol, ijoin, countsf, lencol, comb_hi, comb_lo, hist_W,
         hist_b, fc1_W, fc1_b, fc2_W, fc2_b, fc3_W, fc3_b):
    BB = MLP_BB
    grid = B // BB
    blk = lambda r, c: pl.BlockSpec((BB, c), lambda i: (i, 0))
    full = lambda r, c: pl.BlockSpec((r, c), lambda i: (0, 0))
    vec = lambda n: pl.BlockSpec((n,), lambda i: (0,))
    return pl.pallas_call(
        _mlp_body,
        grid=(grid,),
        in_specs=[
            blk(B, 128), blk(B, 1), blk(B, CW),
            pl.BlockSpec((BB * NI,), lambda i: (i,)), blk(B, 1),
            full(NI, 128), full(NI, 128), full(128, 128), vec(128),
            full(320, 80), vec(80), full(80, 40), vec(40), full(40, 1),
            vec(1),
        ],
        out_specs=[blk(B, 1), blk(B, 1)],
        out_shape=[
            jax.ShapeDtypeStruct((B, 1), jnp.float32),
            jax.ShapeDtypeStruct((B, 1), jnp.float32),
        ],
    )(uemb2, ucol, ijoin, countsf, lencol, comb_hi, comb_lo, hist_W, hist_b,
      fc1_W, fc1_b, fc2_W, fc2_b, fc3_W, fc3_b)


def kernel(user, item, history, length, user_table, item_table, cate_table,
           item_bias, cate_list, hist_W, hist_b, fc1_W, fc1_b, fc2_W, fc2_b,
           fc3_W, fc3_b):
    itp = jnp.pad(item_table, ((0, NI - 1000), (0, 0)))
    clp = jnp.pad(cate_list, (0, NI - 1000)).reshape(NI, 1)
    b16 = jnp.pad(item_bias.reshape(-1, 1), ((0, NI - 1000), (0, 15)))
    combf, comb_hi, comb_lo = _combine(itp, cate_table, clp, b16)
    combined = combf.reshape(NI, CW)

    countsf, ijoin = _sc_hist(history.reshape(-1), item, combined)
    uemb2 = _sc_gather(user, user_table.reshape(USER_PAIRS, 128))

    lencol = length.reshape(B, 1).astype(jnp.float32)
    ucol = user.reshape(B, 1)
    out2, logit2 = _mlp(uemb2, ucol, ijoin, countsf, lencol, comb_hi,
                        comb_lo, hist_W, hist_b, fc1_W, fc1_b, fc2_W, fc2_b,
                        fc3_W, fc3_b)
    return out2[:, 0], logit2[:, 0]


# speedup vs baseline: 1.0479x; 1.0309x over previous
import functools

import jax
import jax.numpy as jnp
from jax import lax
from jax.experimental import pallas as pl
from jax.experimental.pallas import tpu as pltpu
from jax.experimental.pallas import tpu_sc as plsc

B = 4096
L = 50
UD = 64
CW = 256
NI = 1024
BN_SCALE = (1.0 + 1e-3) ** -0.5
NW = 32
RPW = B // NW
HPASS = RPW // 2
USER_PAIRS = 50000
MLP_BB = 1024

_SC_PARAMS = pltpu.CompilerParams(needs_layout_passes=False,
                                  use_tc_tiling_on_sc=False)


def _combine_body(it_ref, ct_ref, cl_ref, b16_ref,
                  out_ref, hi_ref, lo_ref):
    cl = cl_ref[...]
    iota = lax.broadcasted_iota(jnp.int32, (NI, 64), 1)
    oh = (cl == iota).astype(jnp.float32)
    cate_part = jnp.dot(oh, ct_ref[...], preferred_element_type=jnp.float32,
                        precision=lax.Precision.HIGHEST)
    comb = jnp.concatenate(
        [it_ref[...], cate_part, b16_ref[...],
         jnp.zeros((NI, CW - 144), jnp.float32)], axis=1)
    out_ref[...] = comb.reshape(NI * CW)
    c128 = jnp.concatenate([it_ref[...], cate_part], axis=1)
    hi = c128.astype(jnp.bfloat16)
    hi_ref[...] = hi
    lo_ref[...] = (c128 - hi.astype(jnp.float32)).astype(jnp.bfloat16)


_combine = pl.pallas_call(
    _combine_body,
    out_shape=[
        jax.ShapeDtypeStruct((NI * CW,), jnp.float32),
        jax.ShapeDtypeStruct((NI, 128), jnp.bfloat16),
        jax.ShapeDtypeStruct((NI, 128), jnp.bfloat16),
    ],
)


def _sc_hist_body(histf_hbm, item_hbm, comb_hbm, countsf_out, ijoin_out,
                  histf_v, cntf_v, iidx_v, irows_v, sem, sem2):
    wid = lax.axis_index("s") * 2 + lax.axis_index("c")
    base = wid * RPW
    pltpu.sync_copy(item_hbm.at[pl.ds(base, RPW)], iidx_v)
    cp_i = pltpu.async_copy(comb_hbm.at[iidx_v], irows_v, sem2)
    pltpu.sync_copy(histf_hbm.at[pl.ds(base * L, RPW * L)], histf_v)
    zeros16 = jnp.zeros((16,), jnp.float32)
    ones16 = jnp.ones((16,), jnp.float32)
    iota16 = lax.broadcasted_iota(jnp.int32, (16,), 0)
    i50 = iota16 * L
    icnt = iota16 * NI
    for p in range(2):
        def zero_body(i, _):
            for u in range(64):
                cntf_v[pl.ds(i * 1024 + u * 16, 16)] = zeros16
            return 0
        lax.fori_loop(0, HPASS * NI // 1024, zero_body, 0)

        def scat_body(j, _):
            for g in range(HPASS // 16):
                row_off = p * HPASS + g * 16
                hv = plsc.load_gather(histf_v, [i50 + (row_off * L + j)])
                plsc.addupdate_scatter(cntf_v, [icnt + (hv + g * 16 * NI)],
                                       ones16)
            return 0
        lax.fori_loop(0, L, scat_body, 0)
        pltpu.async_copy(
            cntf_v,
            countsf_out.at[pl.ds((base + p * HPASS) * NI, HPASS * NI)],
            sem).wait()
    cp_i.wait()
    pltpu.sync_copy(irows_v, ijoin_out.at[pl.ds(base, RPW)])


def _sc_hist(histf, item, combined):
    mesh = plsc.VectorSubcoreMesh(core_axis_name="c", subcore_axis_name="s")
    f = functools.partial(
        pl.kernel,
        out_type=(
            jax.ShapeDtypeStruct((B * NI,), jnp.float32),
            jax.ShapeDtypeStruct((B, CW), jnp.float32),
        ),
        mesh=mesh,
        compiler_params=_SC_PARAMS,
        scratch_types=[
            pltpu.VMEM((RPW * L,), jnp.int32),
            pltpu.VMEM((HPASS * NI,), jnp.float32),
            pltpu.VMEM((RPW,), jnp.int32),
            pltpu.VMEM((RPW, CW), jnp.float32),
            pltpu.SemaphoreType.DMA,
            pltpu.SemaphoreType.DMA,
        ],
    )(_sc_hist_body)
    return f(histf, item, combined)


def _sc_gather_body(user_hbm, utable_hbm, uemb_out, uidx_v, uidx2_v,
                    urows_v, sem):
    wid = lax.axis_index("s") * 2 + lax.axis_index("c")
    base = wid * RPW
    pltpu.sync_copy(user_hbm.at[pl.ds(base, RPW)], uidx_v)
    for i in range(RPW // 16):
        uidx2_v[pl.ds(i * 16, 16)] = lax.shift_right_logical(
            uidx_v[pl.ds(i * 16, 16)], 1)
    pltpu.async_copy(utable_hbm.at[uidx2_v], urows_v, sem).wait()
    pltpu.sync_copy(urows_v, uemb_out.at[pl.ds(base, RPW)])


def _sc_gather(user, user_table2):
    mesh = plsc.VectorSubcoreMesh(core_axis_name="c", subcore_axis_name="s")
    f = functools.partial(
        pl.kernel,
        out_type=jax.ShapeDtypeStruct((B, 128), jnp.float32),
        mesh=mesh,
        compiler_params=_SC_PARAMS,
        scratch_types=[
            pltpu.VMEM((RPW,), jnp.int32),
            pltpu.VMEM((RPW,), jnp.int32),
            pltpu.VMEM((RPW, 128), jnp.float32),
            pltpu.SemaphoreType.DMA,
        ],
    )(_sc_gather_body)
    return f(user, user_table2)


def _mlp_body(uemb2_ref, ucol_ref, ijoin_ref, cntf_ref, len_ref, hi_ref,
              lo_ref, hW_ref, hb_ref, w1_ref, b1_ref, w2_ref, b2_ref,
              w3_ref, b3_ref, out_ref, logit_ref):
    cnt_bf = cntf_ref[...].reshape(MLP_BB, NI).astype(jnp.bfloat16)
    hist_sum = (jnp.dot(cnt_bf, hi_ref[...], preferred_element_type=jnp.float32)
                + jnp.dot(cnt_bf, lo_ref[...],
                          preferred_element_type=jnp.float32))
    lc = len_ref[...]
    scale = (lc > 0.0).astype(jnp.float32) / lc
    hist_avg = hist_sum * scale
    hist_hid = jnp.dot(hist_avg * BN_SCALE, hW_ref[...],
                       preferred_element_type=jnp.float32) + hb_ref[...].reshape(1, 128)
    ijoin = ijoin_ref[...]
    uemb2 = uemb2_ref[...]
    odd = (ucol_ref[...] & 1) == 1
    uemb = jnp.where(odd, uemb2[:, UD:], uemb2[:, :UD])
    s = jnp.dot(uemb * BN_SCALE, w1_ref[:UD, :],
                preferred_element_type=jnp.float32)
    s += jnp.dot(ijoin[:, :128] * BN_SCALE, w1_ref[UD:UD + 128, :],
                 preferred_element_type=jnp.float32)
    s += jnp.dot(hist_hid * BN_SCALE, w1_ref[UD + 128:, :],
                 preferred_element_type=jnp.float32)
    h1 = jax.nn.sigmoid(s + b1_ref[...].reshape(1, 80))
    h2 = jax.nn.sigmoid(jnp.dot(h1, w2_ref[...],
                                preferred_element_type=jnp.float32)
                        + b2_ref[...].reshape(1, 40))
    out = (jnp.dot(h2, w3_ref[...], preferred_element_type=jnp.float32)
           + b3_ref[...].reshape(1, 1) + ijoin[:, 128:129])
    out_ref[...] = out.reshape(MLP_BB)
    logit_ref[...] = jax.nn.sigmoid(out).reshape(MLP_BB)


def _mlp(uemb2, ucol, ijoin, countsf, lencol, comb_hi, comb_lo, hist_W,
         hist_b, fc1_W, fc1_b, fc2_W, fc2_b, fc3_W, fc3_b):
    BB = MLP_BB
    grid = B // BB
    blk = lambda r, c: pl.BlockSpec((BB, c), lambda i: (i, 0))
    full = lambda r, c: pl.BlockSpec((r, c), lambda i: (0, 0))
    vec = lambda n: pl.BlockSpec((n,), lambda i: (0,))
    return pl.pallas_call(
        _mlp_body,
        grid=(grid,),
        in_specs=[
            blk(B, 128), blk(B, 1), blk(B, CW),
            pl.BlockSpec((BB * NI,), lambda i: (i,)), blk(B, 1),
            full(NI, 128), full(NI, 128), full(128, 128), vec(128),
            full(320, 80), vec(80), full(80, 40), vec(40), full(40, 1),
            vec(1),
        ],
        out_specs=[pl.BlockSpec((BB,), lambda i: (i,)),
                   pl.BlockSpec((BB,), lambda i: (i,))],
        out_shape=[
            jax.ShapeDtypeStruct((B,), jnp.float32),
            jax.ShapeDtypeStruct((B,), jnp.float32),
        ],
    )(uemb2, ucol, ijoin, countsf, lencol, comb_hi, comb_lo, hist_W, hist_b,
      fc1_W, fc1_b, fc2_W, fc2_b, fc3_W, fc3_b)


def kernel(user, item, history, length, user_table, item_table, cate_table,
           item_bias, cate_list, hist_W, hist_b, fc1_W, fc1_b, fc2_W, fc2_b,
           fc3_W, fc3_b):
    itp = jnp.pad(item_table, ((0, NI - 1000), (0, 0)))
    clp = jnp.pad(cate_list, (0, NI - 1000)).reshape(NI, 1)
    b16 = jnp.pad(item_bias.reshape(-1, 1), ((0, NI - 1000), (0, 15)))
    combf, comb_hi, comb_lo = _combine(itp, cate_table, clp, b16)
    combined = combf.reshape(NI, CW)

    countsf, ijoin = _sc_hist(history.reshape(-1), item, combined)
    uemb2 = _sc_gather(user, user_table.reshape(USER_PAIRS, 128))

    lencol = length.reshape(B, 1).astype(jnp.float32)
    ucol = user.reshape(B, 1)
    out1, logit1 = _mlp(uemb2, ucol, ijoin, countsf, lencol, comb_hi,
                        comb_lo, hist_W, hist_b, fc1_W, fc1_b, fc2_W, fc2_b,
                        fc3_W, fc3_b)
    return out1, logit1
